# baseline (speedup 1.0000x reference)
import functools

import jax
import jax.numpy as jnp
import numpy as np
from jax import lax
from jax.experimental import pallas as pl
from jax.experimental.pallas import tpu as pltpu
from jax.experimental.pallas import tpu_sc as plsc

N = 10000
E = 320000
C = 128
NC = 2
NS = 16
NW = NC * NS
EPW = E // NW
K = 40
NCH = EPW // K
RPT = 632
RLAST = N - RPT * (NS - 1)

TBS = 2048
TBG = 512
SIG_S = TBS / 34.0
SIG_O = TBS / 2.0
G_S = TBG / 17.0
SMAX = TBS - 0.001
GMAX = TBG - 0.001
_h1 = 34.0 / TBS
_h2 = 17.0 / TBG
_SVTAB = (1.0 / (1.0 + np.exp(-(-17.0 + (np.arange(TBS) + 0.5) * _h1)))
          ).astype(np.float32)
_GVTAB = np.log1p(np.exp(-((np.arange(TBG) + 0.5) * _h2))).astype(np.float32)

_R = 2000


def _dot(a, b):
    return lax.dot_general(a, b, (((1,), (0,)), ((), ())),
                           precision=lax.Precision.HIGHEST,
                           preferred_element_type=jnp.float32)


def _tables(x, wd, ws, bcat, td_ref, ts_ref):
    pd = _dot(x, wd)
    ps = _dot(x, ws) + bcat
    td_ref[...] = jnp.concatenate([pd[:, :C] * SIG_S, pd[:, C:]], axis=1)
    ts_ref[...] = jnp.concatenate([ps[:, :C] * SIG_S + SIG_O, ps[:, C:]],
                                  axis=1)


def _tables_body(x_ref, wd_ref, ws_ref, b_ref, td_ref, ts_ref):
    _tables(x_ref[...], wd_ref[...], ws_ref[...], b_ref[...], td_ref, ts_ref)


def _combine_tables_body(a0_ref, a1_ref, x_ref, wd_ref, ws_ref, b_ref,
                         x1_ref, td_ref, ts_ref):
    x1 = jnp.maximum(a0_ref[...] + a1_ref[...] + x_ref[...], 0.0)
    x1_ref[...] = x1
    _tables(x1, wd_ref[...], ws_ref[...], b_ref[...], td_ref, ts_ref)


def _final_body(a0_ref, a1_ref, x_ref, x2_ref, ls_ref):
    x2 = jnp.maximum(a0_ref[...] + a1_ref[...] + x_ref[...], 0.0)
    x2_ref[...] = x2
    m = jnp.max(x2, axis=1, keepdims=True)
    sh = x2 - m
    ls_ref[...] = sh - jnp.log(jnp.sum(jnp.exp(sh), axis=1, keepdims=True))


def _tables_call(x, wd, ws, b):
    return pl.pallas_call(
        _tables_body,
        grid=(N // _R,),
        in_specs=[pl.BlockSpec((_R, C), lambda i: (i, 0)),
                  pl.BlockSpec((C, 2 * C), lambda i: (0, 0)),
                  pl.BlockSpec((C, 2 * C), lambda i: (0, 0)),
                  pl.BlockSpec((1, 2 * C), lambda i: (0, 0))],
        out_specs=[pl.BlockSpec((_R, 2 * C), lambda i: (i, 0))] * 2,
        out_shape=[jax.ShapeDtypeStruct((N, 2 * C), jnp.float32)] * 2,
    )(x, wd, ws, b)


def _combine_tables_call(a0, a1, x, wd, ws, b):
    return pl.pallas_call(
        _combine_tables_body,
        grid=(N // _R,),
        in_specs=[pl.BlockSpec((_R, C), lambda i: (i, 0)),
                  pl.BlockSpec((_R, C), lambda i: (i, 0)),
                  pl.BlockSpec((_R, C), lambda i: (i, 0)),
                  pl.BlockSpec((C, 2 * C), lambda i: (0, 0)),
                  pl.BlockSpec((C, 2 * C), lambda i: (0, 0)),
                  pl.BlockSpec((1, 2 * C), lambda i: (0, 0))],
        out_specs=[pl.BlockSpec((_R, C), lambda i: (i, 0)),
                   pl.BlockSpec((_R, 2 * C), lambda i: (i, 0)),
                   pl.BlockSpec((_R, 2 * C), lambda i: (i, 0))],
        out_shape=[jax.ShapeDtypeStruct((N, C), jnp.float32),
                   jax.ShapeDtypeStruct((N, 2 * C), jnp.float32),
                   jax.ShapeDtypeStruct((N, 2 * C), jnp.float32)],
    )(a0, a1, x, wd, ws, b)


def _final_call(a0, a1, x):
    return pl.pallas_call(
        _final_body,
        grid=(N // _R,),
        in_specs=[pl.BlockSpec((_R, C), lambda i: (i, 0))] * 3,
        out_specs=[pl.BlockSpec((_R, C), lambda i: (i, 0))] * 2,
        out_shape=[jax.ShapeDtypeStruct((N, C), jnp.float32)] * 2,
    )(a0, a1, x)


@functools.partial(
    pl.kernel,
    out_type=jax.ShapeDtypeStruct((NC, N, C), jnp.float32),
    mesh=plsc.VectorSubcoreMesh(core_axis_name="c", subcore_axis_name="s",
                                num_cores=NC, num_subcores=NS),
    compiler_params=pltpu.CompilerParams(needs_layout_passes=False),
    scratch_types=[
        pltpu.VMEM_SHARED((N, C), jnp.float32),
        pltpu.VMEM((1, 2, K), jnp.int32),
        pltpu.VMEM((1, 2, K), jnp.int32),
        pltpu.VMEM((K, 2 * C), jnp.float32),
        pltpu.VMEM((K, 2 * C), jnp.float32),
        pltpu.VMEM((K, 2 * C), jnp.float32),
        pltpu.VMEM((K, 2 * C), jnp.float32),
        pltpu.VMEM((K, C), jnp.float32),
        pltpu.VMEM((TBS,), jnp.float32),
        pltpu.VMEM((TBG,), jnp.float32),
        pltpu.SemaphoreType.DMA,
        pltpu.SemaphoreType.DMA,
    ],
)
def _edge_kernel(td, ts, idxe, th0, th1, out, acc, idx0, idx1,
                 bufd0, bufd1, bufs0, bufs1, msg, tsv, tgv, sem, ixsem):
    c = lax.axis_index("c")
    s = lax.axis_index("s")
    t = s * NC + c
    idxv = (idx0, idx1)
    bufd = (bufd0, bufd1)
    bufs = (bufs0, bufs1)
    zero = jnp.zeros((16,), jnp.float32)
    pltpu.sync_copy(th0, tsv)
    pltpu.sync_copy(th1, tgv)

    def zrow(i, carry):
        for j in range(C // 16):
            msg[i, pl.ds(j * 16, 16)] = zero
        return carry

    lax.fori_loop(0, K, zrow, 0)
    zsrc = msg

    @pl.when(s < NS - 1)
    def _():
        for q in range(RPT // K):
            pltpu.sync_copy(zsrc, acc.at[pl.ds(s * RPT + q * K, K)])
        pltpu.sync_copy(msg.at[pl.ds(0, RPT - (RPT // K) * K)],
                        acc.at[pl.ds(s * RPT + (RPT // K) * K,
                                     RPT - (RPT // K) * K)])

    @pl.when(s == NS - 1)
    def _():
        for q in range(RLAST // K):
            pltpu.sync_copy(zsrc, acc.at[pl.ds((NS - 1) * RPT + q * K, K)])

    plsc.subcore_barrier()

    def _idx_start(slot, row):
        pltpu.async_copy(idxe.at[pl.ds(row, 1)], idxv[slot], ixsem)

    def _idx_wait(slot, row):
        pltpu.make_async_copy(idxe.at[pl.ds(row, 1)], idxv[slot],
                              ixsem).wait()

    def _gather_start(slot):
        pltpu.async_copy(td.at[idxv[slot].at[0, 0]], bufd[slot], sem)
        pltpu.async_copy(ts.at[idxv[slot].at[0, 1]], bufs[slot], sem)

    def _gather_wait(slot):
        pltpu.make_async_copy(td.at[idxv[slot].at[0, 0]], bufd[slot],
                              sem).wait()
        pltpu.make_async_copy(ts.at[idxv[slot].at[0, 1]], bufs[slot],
                              sem).wait()

    def _compute_scatter(slot):
        bd, bs = bufd[slot], bufs[slot]

        def edge(e2, carry2):
            for ee in range(4):
                e = e2 * 4 + ee
                for j in range(C // 16):
                    a = bd[e, pl.ds(j * 16, 16)] + bs[e, pl.ds(j * 16, 16)]
                    b = (bd[e, pl.ds(C + j * 16, 16)]
                         + bs[e, pl.ds(C + j * 16, 16)])
                    ta = jnp.minimum(jnp.maximum(a, 0.0), SMAX)
                    sig = plsc.load_gather(tsv, [ta.astype(jnp.int32)])
                    tb = jnp.minimum(jnp.abs(b) * G_S, GMAX)
                    tail = plsc.load_gather(tgv, [tb.astype(jnp.int32)])
                    sp = jnp.maximum(b, 0.0) + tail
                    msg[e, pl.ds(j * 16, 16)] = sig * sp
            return carry2

        lax.fori_loop(0, K // 4, edge, 0)
        pltpu.sync_copy(msg, acc.at[idxv[slot].at[0, 0]], add=True)

    base = t * NCH
    pltpu.sync_copy(idxe.at[pl.ds(base, 1)], idx0)
    _idx_start(1, base + 1)
    _gather_start(0)

    def outer(g, carry):
        for b in range(2):
            i = g * 2 + b
            ob = 1 - b
            _gather_wait(b)
            if b == 0:
                _idx_wait(ob, base + i + 1)
                _gather_start(ob)
            else:

                @pl.when(g < NCH // 2 - 1)
                def _():
                    _idx_wait(ob, base + i + 1)
                    _gather_start(ob)

            _compute_scatter(b)

            @pl.when(i + 2 < NCH)
            def _():
                _idx_start(b, base + i + 2)
        return carry

    lax.fori_loop(0, NCH // 2, outer, 0)
    plsc.subcore_barrier()

    @pl.when(s < NS - 1)
    def _():
        pltpu.sync_copy(acc.at[pl.ds(s * RPT, RPT)],
                        out.at[c, pl.ds(s * RPT, RPT)])

    @pl.when(s == NS - 1)
    def _():
        pltpu.sync_copy(acc.at[pl.ds((NS - 1) * RPT, RLAST)],
                        out.at[c, pl.ds((NS - 1) * RPT, RLAST)])


def _weights(Wf, bf, Ws, bs):
    wd = jnp.concatenate([Wf[:, :C].T, Ws[:, :C].T], axis=1)
    ws = jnp.concatenate([Wf[:, C:].T, Ws[:, C:].T], axis=1)
    b = jnp.concatenate([bf, bs]).reshape(1, 2 * C)
    return wd, ws, b


def kernel(features, edge_index, Wf1, bf1, Ws1, bs1, Wf2, bf2, Ws2, bs2):
    src = edge_index[0].astype(jnp.int32).reshape(NW * NCH, K)
    dst = edge_index[1].astype(jnp.int32).reshape(NW * NCH, K)
    idx = jnp.stack([dst, src], axis=1)
    wd1, ws1, b1 = _weights(Wf1, bf1, Ws1, bs1)
    wd2, ws2, b2 = _weights(Wf2, bf2, Ws2, bs2)

    tbs = jnp.asarray(_SVTAB)
    tbg = jnp.asarray(_GVTAB)

    td1, ts1 = _tables_call(features, wd1, ws1, b1)
    agg1 = _edge_kernel(td1, ts1, idx, tbs, tbg)
    x1, td2, ts2 = _combine_tables_call(agg1[0], agg1[1], features,
                                        wd2, ws2, b2)
    agg2 = _edge_kernel(td2, ts2, idx, tbs, tbg)
    x2, ls = _final_call(agg2[0], agg2[1], x1)
    return (x2, ls)


# speedup vs baseline: 1.0093x; 1.0093x over previous
import functools

import jax
import jax.numpy as jnp
import numpy as np
from jax import lax
from jax.experimental import pallas as pl
from jax.experimental.pallas import tpu as pltpu
from jax.experimental.pallas import tpu_sc as plsc

N = 10000
E = 320000
C = 128
NC = 2
NS = 16
NW = NC * NS
EPW = E // NW
K = 40
NCH = EPW // K
RPT = 632
RLAST = N - RPT * (NS - 1)

TBS = 2048
TBG = 512
SIG_S = TBS / 34.0
SIG_O = TBS / 2.0
G_S = TBG / 17.0
SMAX = TBS - 0.001
GMAX = TBG - 0.001
_h1 = 34.0 / TBS
_h2 = 17.0 / TBG
_SVTAB = (1.0 / (1.0 + np.exp(-(-17.0 + (np.arange(TBS) + 0.5) * _h1)))
          ).astype(np.float32)
_GVTAB = np.log1p(np.exp(-((np.arange(TBG) + 0.5) * _h2))).astype(np.float32)

_R = 2000


def _dot(a, b):
    return lax.dot_general(a, b, (((1,), (0,)), ((), ())),
                           precision=lax.Precision.HIGHEST,
                           preferred_element_type=jnp.float32)


def _tables(x, wd, ws, bcat, td_ref, ts_ref):
    pd = _dot(x, wd)
    ps = _dot(x, ws) + bcat
    td_ref[...] = jnp.concatenate([pd[:, :C] * SIG_S, pd[:, C:]], axis=1)
    ts_ref[...] = jnp.concatenate([ps[:, :C] * SIG_S + SIG_O, ps[:, C:]],
                                  axis=1)


def _tables_body(x_ref, wd_ref, ws_ref, b_ref, td_ref, ts_ref):
    _tables(x_ref[...], wd_ref[...], ws_ref[...], b_ref[...], td_ref, ts_ref)


def _combine_tables_body(a0_ref, a1_ref, x_ref, wd_ref, ws_ref, b_ref,
                         x1_ref, td_ref, ts_ref):
    x1 = jnp.maximum(a0_ref[...] + a1_ref[...] + x_ref[...], 0.0)
    x1_ref[...] = x1
    _tables(x1, wd_ref[...], ws_ref[...], b_ref[...], td_ref, ts_ref)


def _final_body(a0_ref, a1_ref, x_ref, x2_ref, ls_ref):
    x2 = jnp.maximum(a0_ref[...] + a1_ref[...] + x_ref[...], 0.0)
    x2_ref[...] = x2
    m = jnp.max(x2, axis=1, keepdims=True)
    sh = x2 - m
    ls_ref[...] = sh - jnp.log(jnp.sum(jnp.exp(sh), axis=1, keepdims=True))


def _tables_call(x, wd, ws, b):
    return pl.pallas_call(
        _tables_body,
        grid=(N // _R,),
        in_specs=[pl.BlockSpec((_R, C), lambda i: (i, 0)),
                  pl.BlockSpec((C, 2 * C), lambda i: (0, 0)),
                  pl.BlockSpec((C, 2 * C), lambda i: (0, 0)),
                  pl.BlockSpec((1, 2 * C), lambda i: (0, 0))],
        out_specs=[pl.BlockSpec((_R, 2 * C), lambda i: (i, 0))] * 2,
        out_shape=[jax.ShapeDtypeStruct((N, 2 * C), jnp.float32)] * 2,
    )(x, wd, ws, b)


def _combine_tables_call(a0, a1, x, wd, ws, b):
    return pl.pallas_call(
        _combine_tables_body,
        grid=(N // _R,),
        in_specs=[pl.BlockSpec((_R, C), lambda i: (i, 0)),
                  pl.BlockSpec((_R, C), lambda i: (i, 0)),
                  pl.BlockSpec((_R, C), lambda i: (i, 0)),
                  pl.BlockSpec((C, 2 * C), lambda i: (0, 0)),
                  pl.BlockSpec((C, 2 * C), lambda i: (0, 0)),
                  pl.BlockSpec((1, 2 * C), lambda i: (0, 0))],
        out_specs=[pl.BlockSpec((_R, C), lambda i: (i, 0)),
                   pl.BlockSpec((_R, 2 * C), lambda i: (i, 0)),
                   pl.BlockSpec((_R, 2 * C), lambda i: (i, 0))],
        out_shape=[jax.ShapeDtypeStruct((N, C), jnp.float32),
                   jax.ShapeDtypeStruct((N, 2 * C), jnp.float32),
                   jax.ShapeDtypeStruct((N, 2 * C), jnp.float32)],
    )(a0, a1, x, wd, ws, b)


def _final_call(a0, a1, x):
    return pl.pallas_call(
        _final_body,
        grid=(N // _R,),
        in_specs=[pl.BlockSpec((_R, C), lambda i: (i, 0))] * 3,
        out_specs=[pl.BlockSpec((_R, C), lambda i: (i, 0))] * 2,
        out_shape=[jax.ShapeDtypeStruct((N, C), jnp.float32)] * 2,
    )(a0, a1, x)


@functools.partial(
    pl.kernel,
    out_type=jax.ShapeDtypeStruct((NC, N, C), jnp.float32),
    mesh=plsc.VectorSubcoreMesh(core_axis_name="c", subcore_axis_name="s",
                                num_cores=NC, num_subcores=NS),
    compiler_params=pltpu.CompilerParams(needs_layout_passes=False),
    scratch_types=[
        pltpu.VMEM_SHARED((N, C), jnp.float32),
        pltpu.VMEM((1, 2, K), jnp.int32),
        pltpu.VMEM((1, 2, K), jnp.int32),
        pltpu.VMEM((K, 2 * C), jnp.float32),
        pltpu.VMEM((K, 2 * C), jnp.float32),
        pltpu.VMEM((K, 2 * C), jnp.float32),
        pltpu.VMEM((K, 2 * C), jnp.float32),
        pltpu.VMEM((K, C), jnp.float32),
        pltpu.VMEM((TBS,), jnp.float32),
        pltpu.VMEM((TBG,), jnp.float32),
        pltpu.SemaphoreType.DMA,
        pltpu.SemaphoreType.DMA,
    ],
)
def _edge_kernel(td, ts, idxe, th0, th1, out, acc, idx0, idx1,
                 bufd0, bufd1, bufs0, bufs1, msg, tsv, tgv, sem, ixsem):
    c = lax.axis_index("c")
    s = lax.axis_index("s")
    t = s * NC + c
    idxv = (idx0, idx1)
    bufd = (bufd0, bufd1)
    bufs = (bufs0, bufs1)
    zero = jnp.zeros((16,), jnp.float32)
    pltpu.sync_copy(th0, tsv)
    pltpu.sync_copy(th1, tgv)

    def zrow(i, carry):
        for j in range(C // 16):
            msg[i, pl.ds(j * 16, 16)] = zero
        return carry

    lax.fori_loop(0, K, zrow, 0)
    zsrc = msg

    @pl.when(s < NS - 1)
    def _():
        for q in range(RPT // K):
            pltpu.sync_copy(zsrc, acc.at[pl.ds(s * RPT + q * K, K)])
        pltpu.sync_copy(msg.at[pl.ds(0, RPT - (RPT // K) * K)],
                        acc.at[pl.ds(s * RPT + (RPT // K) * K,
                                     RPT - (RPT // K) * K)])

    @pl.when(s == NS - 1)
    def _():
        for q in range(RLAST // K):
            pltpu.sync_copy(zsrc, acc.at[pl.ds((NS - 1) * RPT + q * K, K)])

    plsc.subcore_barrier()

    def _idx_start(slot, row):
        pltpu.async_copy(idxe.at[pl.ds(row, 1)], idxv[slot], ixsem)

    def _idx_wait(slot, row):
        pltpu.make_async_copy(idxe.at[pl.ds(row, 1)], idxv[slot],
                              ixsem).wait()

    def _gather_start(slot):
        pltpu.async_copy(td.at[idxv[slot].at[0, 0]], bufd[slot], sem)
        pltpu.async_copy(ts.at[idxv[slot].at[0, 1]], bufs[slot], sem)

    def _gather_wait(slot):
        pltpu.make_async_copy(td.at[idxv[slot].at[0, 0]], bufd[slot],
                              sem).wait()
        pltpu.make_async_copy(ts.at[idxv[slot].at[0, 1]], bufs[slot],
                              sem).wait()

    def _compute_scatter(slot):
        bd, bs = bufd[slot], bufs[slot]

        def edge(e2, carry2):
            for ee in range(2):
                e = e2 * 2 + ee
                for j in range(C // 16):
                    a = bd[e, pl.ds(j * 16, 16)] + bs[e, pl.ds(j * 16, 16)]
                    b = (bd[e, pl.ds(C + j * 16, 16)]
                         + bs[e, pl.ds(C + j * 16, 16)])
                    ta = jnp.minimum(jnp.maximum(a, 0.0), SMAX)
                    sig = plsc.load_gather(tsv, [ta.astype(jnp.int32)])
                    tb = jnp.minimum(jnp.abs(b) * G_S, GMAX)
                    tail = plsc.load_gather(tgv, [tb.astype(jnp.int32)])
                    sp = jnp.maximum(b, 0.0) + tail
                    msg[e, pl.ds(j * 16, 16)] = sig * sp
            return carry2

        lax.fori_loop(0, K // 2, edge, 0)
        pltpu.sync_copy(msg, acc.at[idxv[slot].at[0, 0]], add=True)

    base = t * NCH
    pltpu.sync_copy(idxe.at[pl.ds(base, 1)], idx0)
    _idx_start(1, base + 1)
    _gather_start(0)

    def outer(g, carry):
        for b in range(2):
            i = g * 2 + b
            ob = 1 - b
            _gather_wait(b)
            if b == 0:
                _idx_wait(ob, base + i + 1)
                _gather_start(ob)
            else:

                @pl.when(g < NCH // 2 - 1)
                def _():
                    _idx_wait(ob, base + i + 1)
                    _gather_start(ob)

            _compute_scatter(b)

            @pl.when(i + 2 < NCH)
            def _():
                _idx_start(b, base + i + 2)
        return carry

    lax.fori_loop(0, NCH // 2, outer, 0)
    plsc.subcore_barrier()

    @pl.when(s < NS - 1)
    def _():
        pltpu.sync_copy(acc.at[pl.ds(s * RPT, RPT)],
                        out.at[c, pl.ds(s * RPT, RPT)])

    @pl.when(s == NS - 1)
    def _():
        pltpu.sync_copy(acc.at[pl.ds((NS - 1) * RPT, RLAST)],
                        out.at[c, pl.ds((NS - 1) * RPT, RLAST)])


def _weights(Wf, bf, Ws, bs):
    wd = jnp.concatenate([Wf[:, :C].T, Ws[:, :C].T], axis=1)
    ws = jnp.concatenate([Wf[:, C:].T, Ws[:, C:].T], axis=1)
    b = jnp.concatenate([bf, bs]).reshape(1, 2 * C)
    return wd, ws, b


def kernel(features, edge_index, Wf1, bf1, Ws1, bs1, Wf2, bf2, Ws2, bs2):
    src = edge_index[0].astype(jnp.int32).reshape(NW * NCH, K)
    dst = edge_index[1].astype(jnp.int32).reshape(NW * NCH, K)
    idx = jnp.stack([dst, src], axis=1)
    wd1, ws1, b1 = _weights(Wf1, bf1, Ws1, bs1)
    wd2, ws2, b2 = _weights(Wf2, bf2, Ws2, bs2)

    tbs = jnp.asarray(_SVTAB)
    tbg = jnp.asarray(_GVTAB)

    td1, ts1 = _tables_call(features, wd1, ws1, b1)
    agg1 = _edge_kernel(td1, ts1, idx, tbs, tbg)
    x1, td2, ts2 = _combine_tables_call(agg1[0], agg1[1], features,
                                        wd2, ws2, b2)
    agg2 = _edge_kernel(td2, ts2, idx, tbs, tbg)
    x2, ls = _final_call(agg2[0], agg2[1], x1)
    return (x2, ls)
